# baseline (speedup 1.0000x reference)
import functools

import jax
import jax.numpy as jnp
from jax import lax
from jax.experimental import pallas as pl
from jax.experimental.pallas import tpu as pltpu
from jax.experimental.pallas import tpu_sc as plsc

_BLOCK = 6144
_CHAINS = 3
_SUB = _BLOCK // _CHAINS

_NC = 2
_NS = 16
_NW = _NC * _NS
_GCHUNK = 128


def _argmin_fold(s2, rnorm, en, k):
    iota = lax.broadcasted_iota(jnp.int32, s2.shape, 1)
    w = k // 2
    a = (rnorm + en[:, :w]) - s2[:, :w]
    b = (rnorm + en[:, w:]) - s2[:, w:]
    take_b = b < a
    v = jnp.minimum(a, b)
    i = jnp.where(take_b, iota[:, w:], iota[:, :w])
    while w > 128:
        w //= 2
        a, b = v[:, :w], v[:, w:]
        ia, ib = i[:, :w], i[:, w:]
        take_b = b < a
        v = jnp.minimum(a, b)
        i = jnp.where(take_b, ib, ia)
    m = jnp.min(v, axis=1, keepdims=True)
    return jnp.min(jnp.where(v == m, i, k), axis=1)


def _bodyA(z_ref, rn_ref, e0_ref, en0_ref, i0_ref):
    k = e0_ref.shape[0]
    for h in range(_CHAINS):
        sl = pl.ds(h * _SUB, _SUB)
        z = z_ref[sl, :]
        z2 = z + z
        s2 = lax.dot_general(z2, e0_ref[...], (((1,), (1,)), ((), ())),
                             preferred_element_type=jnp.float32)
        idx0 = _argmin_fold(s2, rn_ref[sl, :], en0_ref[...], k)
        i0_ref[sl, :] = idx0[:, None]


def _bodyB(z_ref, q0_ref, e1_ref, en1_ref, i1_ref):
    k = e1_ref.shape[0]
    d = z_ref.shape[1]
    for h in range(_CHAINS):
        sl = pl.ds(h * _SUB, _SUB)
        z = z_ref[sl, :]
        q0 = q0_ref[sl, :d]
        t0 = q0 - z
        qs0 = z + t0
        r1 = z - qs0
        r1_2 = r1 + r1
        rn1 = jnp.sum(r1 * r1, axis=1, keepdims=True)
        s2 = lax.dot_general(r1_2, e1_ref[...], (((1,), (1,)), ((), ())),
                             preferred_element_type=jnp.float32)
        idx1 = _argmin_fold(s2, rn1, en1_ref[...], k)
        i1_ref[sl, :] = idx1[:, None]


def _bodyC(z_ref, q0_ref, q1_ref, q_ref, l0_ref, l1_ref):
    d = z_ref.shape[1]
    z = z_ref[...]
    q0 = q0_ref[:, :d]
    q1 = q1_ref[:, :d]
    t0 = q0 - z
    qs0 = z + t0
    r1 = z - qs0
    t1 = q1 - r1
    qs1 = r1 + t1
    q_ref[...] = qs0 + qs1

    @pl.when(pl.program_id(0) == 0)
    def _():
        l0_ref[...] = jnp.zeros((1, 1), jnp.float32)
        l1_ref[...] = jnp.zeros((1, 1), jnp.float32)

    l0_ref[...] += jnp.sum(t0 * t0).reshape(1, 1)
    l1_ref[...] += jnp.sum(t1 * t1).reshape(1, 1)


def _sc_gather(table, idx3):
    _, nch, width = idx3.shape
    n = _NW * nch * width
    d = table.shape[1]
    per_w = nch * width
    mesh = plsc.VectorSubcoreMesh(core_axis_name="c", subcore_axis_name="s")

    @functools.partial(
        pl.kernel, mesh=mesh,
        out_type=jax.ShapeDtypeStruct((n, d), jnp.float32),
        scratch_types=[
            pltpu.VMEM((nch, width), jnp.int32),
            pltpu.VMEM((width, d), jnp.float32),
            pltpu.VMEM((width, d), jnp.float32),
            pltpu.SemaphoreType.DMA,
            pltpu.SemaphoreType.DMA,
        ],
    )
    def k(table_hbm, idx_hbm, out_hbm, idx_v, rows0, rows1, sem0, sem1):
        wid = lax.axis_index("s") * _NC + lax.axis_index("c")
        obase = wid * per_w
        pltpu.sync_copy(idx_hbm.at[wid], idx_v)
        bufs = (rows0, rows1)
        sems = (sem0, sem1)
        cps = [None, None]
        for c in range(nch + 1):
            if c < nch:
                cps[c & 1] = pltpu.async_copy(
                    table_hbm.at[idx_v.at[c]], bufs[c & 1], sems[c & 1])
            if c >= 1:
                p = (c - 1) & 1
                cps[p].wait()
                pltpu.sync_copy(
                    bufs[p], out_hbm.at[pl.ds(obase + (c - 1) * width, width)])

    return k(table, idx3)


def kernel(z_flat, codebook0, codebook1):
    n, d = z_flat.shape
    k = codebook0.shape[0]
    rn = jnp.sum(z_flat ** 2, axis=1, keepdims=True)
    en0 = jnp.sum(codebook0 ** 2, axis=1).reshape(1, k)
    en1 = jnp.sum(codebook1 ** 2, axis=1).reshape(1, k)

    row = lambda i: (i, 0)
    rep = lambda i: (0, 0)
    grid = (n // _BLOCK,)

    pad = jnp.zeros((k, 128 - d), jnp.float32)
    table0 = jnp.concatenate([codebook0, pad], axis=1)
    table1 = jnp.concatenate([codebook1, pad], axis=1)

    def stage_a(z_h, rn_h):
        nh = z_h.shape[0]
        return pl.pallas_call(
            _bodyA,
            grid=(nh // _BLOCK,),
            in_specs=[
                pl.BlockSpec((_BLOCK, d), row),
                pl.BlockSpec((_BLOCK, 1), row),
                pl.BlockSpec((k, d), rep),
                pl.BlockSpec((1, k), rep),
            ],
            out_specs=pl.BlockSpec((_BLOCK, 1), row),
            out_shape=jax.ShapeDtypeStruct((nh, 1), jnp.int32),
        )(z_h, rn_h, codebook0, en0)

    def stage_b(z_h, q0_h):
        nh = z_h.shape[0]
        return pl.pallas_call(
            _bodyB,
            grid=(nh // _BLOCK,),
            in_specs=[
                pl.BlockSpec((_BLOCK, d), row),
                pl.BlockSpec((_BLOCK, 128), row),
                pl.BlockSpec((k, d), rep),
                pl.BlockSpec((1, k), rep),
            ],
            out_specs=pl.BlockSpec((_BLOCK, 1), row),
            out_shape=jax.ShapeDtypeStruct((nh, 1), jnp.int32),
        )(z_h, q0_h, codebook1, en1)

    def stage_c(z_h, q0_h, q1_h):
        nh = z_h.shape[0]
        return pl.pallas_call(
            _bodyC,
            grid=(nh // _BLOCK,),
            in_specs=[
                pl.BlockSpec((_BLOCK, d), row),
                pl.BlockSpec((_BLOCK, 128), row),
                pl.BlockSpec((_BLOCK, 128), row),
            ],
            out_specs=[
                pl.BlockSpec((_BLOCK, d), row),
                pl.BlockSpec((1, 1), rep),
                pl.BlockSpec((1, 1), rep),
            ],
            out_shape=[
                jax.ShapeDtypeStruct((nh, d), jnp.float32),
                jax.ShapeDtypeStruct((1, 1), jnp.float32),
                jax.ShapeDtypeStruct((1, 1), jnp.float32),
            ],
        )(z_h, q0_h, q1_h)

    def gather(table, i_h):
        nh = i_h.shape[0]
        return _sc_gather(table, i_h.reshape(_NW, nh // (_NW * _GCHUNK), _GCHUNK))

    nchunks = 3
    nc_rows = n // nchunks
    i0s, i1s, outs = [], [], []
    for c in range(nchunks):
        sl = slice(c * nc_rows, (c + 1) * nc_rows)
        z_h, rn_h = z_flat[sl], rn[sl]
        i0_h = stage_a(z_h, rn_h)
        q0_h = gather(table0, i0_h)
        i1_h = stage_b(z_h, q0_h)
        q1_h = gather(table1, i1_h)
        outs.append(stage_c(z_h, q0_h, q1_h))
        i0s.append(i0_h)
        i1s.append(i1_h)

    q = jnp.concatenate([o[0] for o in outs], axis=0)
    i0 = jnp.concatenate(i0s, axis=0)
    i1 = jnp.concatenate(i1s, axis=0)
    l0 = outs[0][1] + outs[1][1] + outs[2][1]
    l1 = outs[0][2] + outs[1][2] + outs[2][2]

    nd = jnp.float32(n * d)
    m0 = l0[0, 0] / nd
    m1 = l1[0, 0] / nd
    loss0 = m0 + 0.25 * m0
    loss1 = m1 + 0.25 * m1
    total = loss0 + loss1
    return (total, q, i0.reshape(n), i1.reshape(n))


# speedup vs baseline: 1.1845x; 1.1845x over previous
import functools

import jax
import jax.numpy as jnp
from jax import lax
from jax.experimental import pallas as pl
from jax.experimental.pallas import tpu as pltpu
from jax.experimental.pallas import tpu_sc as plsc

_BLOCK = 6144
_CHAINS = 3
_SUB = _BLOCK // _CHAINS

_NC = 2
_NS = 16
_NW = _NC * _NS
_GCHUNK = 128


def _argmin_fold(s2, rnorm, en, k):
    iota = lax.broadcasted_iota(jnp.int32, s2.shape, 1)
    w = k // 2
    a = (rnorm + en[:, :w]) - s2[:, :w]
    b = (rnorm + en[:, w:]) - s2[:, w:]
    take_b = b < a
    v = jnp.minimum(a, b)
    i = jnp.where(take_b, iota[:, w:], iota[:, :w])
    while w > 128:
        w //= 2
        a, b = v[:, :w], v[:, w:]
        ia, ib = i[:, :w], i[:, w:]
        take_b = b < a
        v = jnp.minimum(a, b)
        i = jnp.where(take_b, ib, ia)
    m = jnp.min(v, axis=1, keepdims=True)
    return jnp.min(jnp.where(v == m, i, k), axis=1)


def _bodyA(z_ref, rn_ref, e0_ref, en0_ref, i0_ref):
    k = e0_ref.shape[0]
    for h in range(_CHAINS):
        sl = pl.ds(h * _SUB, _SUB)
        z = z_ref[sl, :]
        z2 = z + z
        s2 = lax.dot_general(z2, e0_ref[...], (((1,), (1,)), ((), ())),
                             preferred_element_type=jnp.float32)
        idx0 = _argmin_fold(s2, rn_ref[sl, :], en0_ref[...], k)
        i0_ref[sl, :] = idx0[:, None]


def _bodyB(z_ref, q0_ref, e1_ref, en1_ref, i1_ref):
    k = e1_ref.shape[0]
    d = z_ref.shape[1]
    for h in range(_CHAINS):
        sl = pl.ds(h * _SUB, _SUB)
        z = z_ref[sl, :]
        q0 = q0_ref[sl, :d]
        t0 = q0 - z
        qs0 = z + t0
        r1 = z - qs0
        r1_2 = r1 + r1
        rn1 = jnp.sum(r1 * r1, axis=1, keepdims=True)
        s2 = lax.dot_general(r1_2, e1_ref[...], (((1,), (1,)), ((), ())),
                             preferred_element_type=jnp.float32)
        idx1 = _argmin_fold(s2, rn1, en1_ref[...], k)
        i1_ref[sl, :] = idx1[:, None]


def _bodyC(z_ref, q0_ref, q1_ref, q_ref, l0_ref, l1_ref):
    d = z_ref.shape[1]
    z = z_ref[...]
    q0 = q0_ref[:, :d]
    q1 = q1_ref[:, :d]
    t0 = q0 - z
    qs0 = z + t0
    r1 = z - qs0
    t1 = q1 - r1
    qs1 = r1 + t1
    q_ref[...] = qs0 + qs1

    @pl.when(pl.program_id(0) == 0)
    def _():
        l0_ref[...] = jnp.zeros((1, 1), jnp.float32)
        l1_ref[...] = jnp.zeros((1, 1), jnp.float32)

    l0_ref[...] += jnp.sum(t0 * t0).reshape(1, 1)
    l1_ref[...] += jnp.sum(t1 * t1).reshape(1, 1)


def _sc_gather(table, idx3):
    _, nch, width = idx3.shape
    n = _NW * nch * width
    d = table.shape[1]
    per_w = nch * width
    mesh = plsc.VectorSubcoreMesh(core_axis_name="c", subcore_axis_name="s")

    @functools.partial(
        pl.kernel, mesh=mesh,
        out_type=jax.ShapeDtypeStruct((n, d), jnp.float32),
        scratch_types=[
            pltpu.VMEM((nch, width), jnp.int32),
            pltpu.VMEM((width, d), jnp.float32),
            pltpu.VMEM((width, d), jnp.float32),
            pltpu.SemaphoreType.DMA,
            pltpu.SemaphoreType.DMA,
        ],
    )
    def k(table_hbm, idx_hbm, out_hbm, idx_v, rows0, rows1, sem0, sem1):
        wid = lax.axis_index("s") * _NC + lax.axis_index("c")
        obase = wid * per_w
        pltpu.sync_copy(idx_hbm.at[wid], idx_v)
        bufs = (rows0, rows1)
        sems = (sem0, sem1)
        cps = [None, None]
        for c in range(nch + 1):
            if c < nch:
                cps[c & 1] = pltpu.async_copy(
                    table_hbm.at[idx_v.at[c]], bufs[c & 1], sems[c & 1])
            if c >= 1:
                p = (c - 1) & 1
                cps[p].wait()
                pltpu.sync_copy(
                    bufs[p], out_hbm.at[pl.ds(obase + (c - 1) * width, width)])

    return k(table, idx3)


def kernel(z_flat, codebook0, codebook1):
    n, d = z_flat.shape
    k = codebook0.shape[0]
    rn = jnp.sum(z_flat ** 2, axis=1, keepdims=True)
    en0 = jnp.sum(codebook0 ** 2, axis=1).reshape(1, k)
    en1 = jnp.sum(codebook1 ** 2, axis=1).reshape(1, k)

    row = lambda i: (i, 0)
    rep = lambda i: (0, 0)
    grid = (n // _BLOCK,)

    pad = jnp.zeros((k, 128 - d), jnp.float32)
    table0 = jnp.concatenate([codebook0, pad], axis=1)
    table1 = jnp.concatenate([codebook1, pad], axis=1)

    def stage_a(z_h, rn_h):
        nh = z_h.shape[0]
        return pl.pallas_call(
            _bodyA,
            grid=(nh // _BLOCK,),
            in_specs=[
                pl.BlockSpec((_BLOCK, d), row),
                pl.BlockSpec((_BLOCK, 1), row),
                pl.BlockSpec((k, d), rep),
                pl.BlockSpec((1, k), rep),
            ],
            out_specs=pl.BlockSpec((_BLOCK, 1), row),
            out_shape=jax.ShapeDtypeStruct((nh, 1), jnp.int32),
        )(z_h, rn_h, codebook0, en0)

    def stage_b(z_h, q0_h):
        nh = z_h.shape[0]
        return pl.pallas_call(
            _bodyB,
            grid=(nh // _BLOCK,),
            in_specs=[
                pl.BlockSpec((_BLOCK, d), row),
                pl.BlockSpec((_BLOCK, 128), row),
                pl.BlockSpec((k, d), rep),
                pl.BlockSpec((1, k), rep),
            ],
            out_specs=pl.BlockSpec((_BLOCK, 1), row),
            out_shape=jax.ShapeDtypeStruct((nh, 1), jnp.int32),
        )(z_h, q0_h, codebook1, en1)

    def stage_c(z_h, q0_h, q1_h):
        nh = z_h.shape[0]
        return pl.pallas_call(
            _bodyC,
            grid=(nh // _BLOCK,),
            in_specs=[
                pl.BlockSpec((_BLOCK, d), row),
                pl.BlockSpec((_BLOCK, 128), row),
                pl.BlockSpec((_BLOCK, 128), row),
            ],
            out_specs=[
                pl.BlockSpec((_BLOCK, d), row),
                pl.BlockSpec((1, 1), rep),
                pl.BlockSpec((1, 1), rep),
            ],
            out_shape=[
                jax.ShapeDtypeStruct((nh, d), jnp.float32),
                jax.ShapeDtypeStruct((1, 1), jnp.float32),
                jax.ShapeDtypeStruct((1, 1), jnp.float32),
            ],
        )(z_h, q0_h, q1_h)

    def gather(table, i_h):
        nh = i_h.shape[0]
        return _sc_gather(table, i_h.reshape(_NW, nh // (_NW * _GCHUNK), _GCHUNK))

    i0 = stage_a(z_flat, rn)
    q0 = gather(table0, i0)
    i1 = stage_b(z_flat, q0)
    q1 = gather(table1, i1)
    q, l0, l1 = stage_c(z_flat, q0, q1)

    nd = jnp.float32(n * d)
    m0 = l0[0, 0] / nd
    m1 = l1[0, 0] / nd
    loss0 = m0 + 0.25 * m0
    loss1 = m1 + 0.25 * m1
    total = loss0 + loss1
    return (total, q, i0.reshape(n), i1.reshape(n))
